# Initial kernel scaffold; baseline (speedup 1.0000x reference)
import jax
import jax.numpy as jnp
from jax.experimental import pallas as pl


def kernel(x, edge_index, W, gamma, beta):
    raise NotImplementedError("write your pallas kernel here")


# speedup vs baseline: 12.9920x; 12.9920x over previous
import functools

import jax
import jax.numpy as jnp
from jax import lax
from jax.experimental import pallas as pl
from jax.experimental.pallas import tpu as pltpu
from jax.experimental.pallas import tpu_sc as plsc

N = 10000
E = 320000
D = 128

NC = 2
NS = 16
NW = NC * NS
EPW = E // NW
CH = 100
NCH = EPW // CH
NP = 10112
RPS = NP // NS
CW = 16


def _sc_feats(x, src, dst, zinit):
  mesh = plsc.VectorSubcoreMesh(core_axis_name="c", subcore_axis_name="s")

  @functools.partial(
      pl.kernel,
      out_type=jax.ShapeDtypeStruct((NC, NP, D), jnp.float32),
      mesh=mesh,
      scratch_types=[
          pltpu.VMEM((NCH, CH), jnp.int32),
          pltpu.VMEM((NCH, CH), jnp.int32),
          pltpu.VMEM((CH, D), jnp.float32),
          pltpu.VMEM((CH, D), jnp.float32),
          pltpu.VMEM_SHARED((NP, D), jnp.float32),
          pltpu.SemaphoreType.DMA,
          pltpu.SemaphoreType.DMA,
      ],
      compiler_params=pltpu.CompilerParams(use_tc_tiling_on_sc=False),
  )
  def k(x_hbm, src_hbm, dst_hbm, z_hbm, out_hbm,
        src_v, dst_v, buf0, buf1, acc_sh, sem0, sem1):
    cid = lax.axis_index("c")
    sid = lax.axis_index("s")
    wid = sid * NC + cid

    pltpu.sync_copy(z_hbm.at[pl.ds(sid * RPS, RPS)],
                    acc_sh.at[pl.ds(sid * RPS, RPS)])
    pltpu.sync_copy(src_hbm.at[wid], src_v)
    pltpu.sync_copy(dst_hbm.at[wid], dst_v)
    plsc.subcore_barrier()

    def gather(c, buf, sem):
      pltpu.async_copy(x_hbm.at[src_v.at[c]], buf, sem)

    def gwait(c, buf, sem):
      pltpu.make_async_copy(x_hbm.at[src_v.at[c]], buf, sem).wait()

    def scatter(c, buf):
      pltpu.sync_copy(buf, acc_sh.at[dst_v.at[c]], add=True)

    gather(0, buf0, sem0)

    def body(t, carry):
      c = 2 * t
      gather(c + 1, buf1, sem1)
      gwait(c, buf0, sem0)
      scatter(c, buf0)
      gather(c + 2, buf0, sem0)
      gwait(c + 1, buf1, sem1)
      scatter(c + 1, buf1)
      return carry

    lax.fori_loop(0, NCH // 2 - 1, body, 0)
    c = NCH - 2
    gather(c + 1, buf1, sem1)
    gwait(c, buf0, sem0)
    scatter(c, buf0)
    gwait(c + 1, buf1, sem1)
    scatter(c + 1, buf1)

    plsc.subcore_barrier()
    pltpu.sync_copy(acc_sh.at[pl.ds(sid * RPS, RPS)],
                    out_hbm.at[cid, pl.ds(sid * RPS, RPS)])

  return k(x, src, dst, zinit)


def _sc_counts(dst, ones_rows, zinit):
  mesh = plsc.VectorSubcoreMesh(core_axis_name="c", subcore_axis_name="s")

  @functools.partial(
      pl.kernel,
      out_type=jax.ShapeDtypeStruct((NC, NP, CW), jnp.float32),
      mesh=mesh,
      scratch_types=[
          pltpu.VMEM((NCH, CH), jnp.int32),
          pltpu.VMEM((CH, CW), jnp.float32),
          pltpu.VMEM_SHARED((NP, CW), jnp.float32),
          pltpu.SemaphoreType.DMA,
      ],
      compiler_params=pltpu.CompilerParams(use_tc_tiling_on_sc=False),
  )
  def k(dst_hbm, ones_hbm, z_hbm, out_hbm, dst_v, ones_v, acc_sh, sem):
    cid = lax.axis_index("c")
    sid = lax.axis_index("s")
    wid = sid * NC + cid

    pltpu.sync_copy(z_hbm.at[pl.ds(sid * RPS, RPS)],
                    acc_sh.at[pl.ds(sid * RPS, RPS)])
    pltpu.sync_copy(dst_hbm.at[wid], dst_v)
    pltpu.sync_copy(ones_hbm, ones_v)
    plsc.subcore_barrier()

    def fire(c, carry):
      pltpu.async_copy(ones_v, acc_sh.at[dst_v.at[c]], sem, add=True)
      return carry

    def drain(c, carry):
      pltpu.make_async_copy(ones_v, acc_sh.at[dst_v.at[c]], sem).wait()
      return carry

    lax.fori_loop(0, NCH, fire, 0)
    lax.fori_loop(0, NCH, drain, 0)

    plsc.subcore_barrier()
    pltpu.sync_copy(acc_sh.at[pl.ds(sid * RPS, RPS)],
                    out_hbm.at[cid, pl.ds(sid * RPS, RPS)])

  return k(dst, ones_rows, zinit)


def _tc_dense(acc, cacc, w_t, gamma, beta):
  def body(acc_ref, c_ref, w_ref, g_ref, b_ref, out_ref):
    s = acc_ref[0, :N] + acc_ref[1, :N]
    cnt = c_ref[0, :N, 0:1] + c_ref[1, :N, 0:1]
    mean = s / jnp.maximum(cnt, 1.0)
    h = jnp.dot(mean, w_ref[...], preferred_element_type=jnp.float32)
    h = jnp.where(h > 0, h, jnp.exp(jnp.minimum(h, 0.0)) - 1.0)
    mu = jnp.mean(h, axis=0, keepdims=True)
    xc = h - mu
    var = jnp.mean(xc * xc, axis=0, keepdims=True)
    out_ref[...] = g_ref[...] * (xc * lax.rsqrt(var + 1e-5)) + b_ref[...]

  return pl.pallas_call(
      body,
      out_shape=jax.ShapeDtypeStruct((N, D), jnp.float32),
  )(acc, cacc, w_t, gamma, beta)


def kernel(x, edge_index, W, gamma, beta):
  src = edge_index[0].reshape(NW, NCH, CH)
  dst = edge_index[1].reshape(NW, NCH, CH)
  zinit = jnp.zeros((NP, D), jnp.float32)
  zinit_c = jnp.zeros((NP, CW), jnp.float32)
  ones_rows = jnp.ones((CH, CW), jnp.float32)
  acc = _sc_feats(x, src, dst, zinit)
  cacc = _sc_counts(dst, ones_rows, zinit_c)
  return _tc_dense(acc, cacc, W.T, gamma.reshape(1, D), beta.reshape(1, D))
